# baseline (speedup 1.0000x reference)
import functools

import jax
import jax.numpy as jnp
from jax import lax
from jax.experimental import pallas as pl
from jax.experimental.pallas import tpu as pltpu
from jax.experimental.pallas import tpu_sc as plsc

EMBED_DIM = 32
BATCH = 16384
FIELDS = 26
NUM_CORES = 2
NUM_SUBCORES = 16
NUM_WORKERS = NUM_CORES * NUM_SUBCORES
BATCH_PER_WORKER = BATCH // NUM_WORKERS
BBLK = 128
BLKS = BATCH_PER_WORKER // BBLK
NCHUNKS = FIELDS * BLKS
L = 16


def _build():
    mesh = plsc.VectorSubcoreMesh(core_axis_name="c", subcore_axis_name="s")

    scratch = (
        [pltpu.VMEM((BBLK,), jnp.int32) for _ in range(2)]
        + [pltpu.VMEM((BBLK,), jnp.int32) for _ in range(2)]
        + [pltpu.VMEM((BBLK,), jnp.int32) for _ in range(2)]
        + [pltpu.VMEM((BBLK, 128), jnp.float32) for _ in range(2)]
        + [pltpu.VMEM((EMBED_DIM, BBLK), jnp.float32) for _ in range(2)]
        + [pltpu.SemaphoreType.DMA for _ in range(4)]
    )

    @functools.partial(
        pl.kernel,
        mesh=mesh,
        out_type=jax.ShapeDtypeStruct((FIELDS, EMBED_DIM, BATCH), jnp.float32),
        scratch_types=scratch,
        compiler_params=pltpu.CompilerParams(
            use_tc_tiling_on_sc=True, needs_layout_passes=False
        ),
    )
    def body(xl_ref, w_ref, out_ref, *s):
        idx = s[0:2]
        big = s[2:4]
        rem = s[4:6]
        g = s[6:8]
        ot = s[8:10]
        gsem = s[10:12]
        wsem = s[12:14]

        wid = lax.axis_index("s") * NUM_CORES + lax.axis_index("c")
        b0 = wid * BATCH_PER_WORKER

        def chunk_off(c):
            f = c // BLKS
            blk = lax.rem(c, BLKS)
            return f, f * BATCH + b0 + blk * BBLK

        def load_and_fire(c, p):
            _, off = chunk_off(c)
            pltpu.sync_copy(xl_ref.at[pl.ds(off, BBLK)], idx[p])
            def split(v, _):
                iv = idx[p][pl.ds(v * L, L)]
                big[p][pl.ds(v * L, L)] = iv >> 2
                rem[p][pl.ds(v * L, L)] = (iv & 3) * EMBED_DIM
                return _
            lax.fori_loop(0, BBLK // L, split, 0)
            pltpu.async_copy(w_ref.at[big[p]], g[p], gsem[p])

        def wait_gather(p):
            pltpu.make_async_copy(w_ref.at[big[p]], g[p], gsem[p]).wait()

        def transpose_select(p):
            def per_c(c, _):
                def per_blk(k, __):
                    jv = k * L + lax.iota(jnp.int32, L)
                    cv = rem[p][pl.ds(k * L, L)] + c
                    vals = plsc.load_gather(g[p], [jv, cv])
                    ot[p][c, pl.ds(k * L, L)] = vals
                    return __
                return lax.fori_loop(0, BBLK // L, per_blk, _)
            lax.fori_loop(0, EMBED_DIM, per_c, 0)

        def fire_write(c, p):
            f, off = chunk_off(c)
            bb = off - f * BATCH
            pltpu.async_copy(ot[p], out_ref.at[f, :, pl.ds(bb, BBLK)], wsem[p])

        def wait_write(c, p):
            f, off = chunk_off(c)
            bb = off - f * BATCH
            pltpu.make_async_copy(
                ot[p], out_ref.at[f, :, pl.ds(bb, BBLK)], wsem[p]
            ).wait()

        load_and_fire(0, 0)

        def round_body(r, carry):
            for k in range(2):
                c = 2 * r + k
                wait_gather(k)

                @pl.when(c + 1 < NCHUNKS)
                def _():
                    load_and_fire(c + 1, 1 - k)

                @pl.when(c >= 2)
                def _():
                    wait_write(c - 2, k)

                transpose_select(k)
                fire_write(c, k)
            return carry

        lax.fori_loop(0, NCHUNKS // 2, round_body, 0)
        wait_write(NCHUNKS - 2, 0)
        wait_write(NCHUNKS - 1, 1)

    return body


_gather_kernel = _build()


def kernel(x, weight):
    xl = x.T.reshape(-1)
    w4 = weight.reshape(250000, 128)
    out = _gather_kernel(xl, w4)
    return out.transpose(2, 0, 1)


# speedup vs baseline: 1.3745x; 1.3745x over previous
import functools

import jax
import jax.numpy as jnp
from jax import lax
from jax.experimental import pallas as pl
from jax.experimental.pallas import tpu as pltpu
from jax.experimental.pallas import tpu_sc as plsc

EMBED_DIM = 32
BATCH = 16384
FIELDS = 26
NUM_CORES = 2
NUM_SUBCORES = 16
NUM_WORKERS = NUM_CORES * NUM_SUBCORES
BATCH_PER_WORKER = BATCH // NUM_WORKERS
NCHUNKS = FIELDS
NBUF = 6
D1 = 3
D2 = NBUF - D1


def _build():
    mesh = plsc.VectorSubcoreMesh(core_axis_name="c", subcore_axis_name="s")

    scratch = (
        [pltpu.VMEM((BATCH_PER_WORKER,), jnp.int32) for _ in range(NBUF)]
        + [pltpu.VMEM((BATCH_PER_WORKER, EMBED_DIM), jnp.float32) for _ in range(NBUF)]
        + [pltpu.SemaphoreType.DMA for _ in range(2 * NBUF)]
    )

    @functools.partial(
        pl.kernel,
        mesh=mesh,
        out_type=jax.ShapeDtypeStruct((BATCH, FIELDS, EMBED_DIM), jnp.float32),
        scratch_types=scratch,
        compiler_params=pltpu.CompilerParams(use_tc_tiling_on_sc=False),
    )
    def body(xt_ref, w_ref, out_ref, *s):
        idx = s[0:NBUF]
        rows = s[NBUF:2 * NBUF]
        gsem = s[2 * NBUF:3 * NBUF]
        wsem = s[3 * NBUF:4 * NBUF]

        wid = lax.axis_index("s") * NUM_CORES + lax.axis_index("c")
        b0 = wid * BATCH_PER_WORKER

        def fire_gather(f, b):
            pltpu.sync_copy(xt_ref.at[f, pl.ds(b0, BATCH_PER_WORKER)], idx[b])
            pltpu.async_copy(w_ref.at[idx[b]], rows[b], gsem[b])

        def wait_gather(f, b):
            pltpu.make_async_copy(w_ref.at[idx[b]], rows[b], gsem[b]).wait()

        def fire_write(f, b):
            pltpu.async_copy(
                rows[b], out_ref.at[pl.ds(b0, BATCH_PER_WORKER), f, :], wsem[b]
            )

        def wait_write(f, b):
            pltpu.make_async_copy(
                rows[b], out_ref.at[pl.ds(b0, BATCH_PER_WORKER), f, :], wsem[b]
            ).wait()

        for f in range(D1):
            fire_gather(f, f % NBUF)
        for f in range(NCHUNKS):
            b = f % NBUF
            wait_gather(f, b)
            fire_write(f, b)
            f2 = f + D1
            if f2 < NCHUNKS:
                b2 = f2 % NBUF
                if f2 - NBUF >= 0:
                    wait_write(f2 - NBUF, b2)
                fire_gather(f2, b2)
        for f in range(NCHUNKS - NBUF, NCHUNKS):
            wait_write(f, f % NBUF)

    return body


_gather_kernel = _build()


def kernel(x, weight):
    return _gather_kernel(x.T, weight)
